# baseline (speedup 1.0000x reference)
import functools

import jax
import jax.numpy as jnp
from jax import lax
from jax.experimental import pallas as pl
from jax.experimental.pallas import tpu as pltpu
from jax.experimental.pallas import tpu_sc as plsc

_VOCAB = 100000
_EMBED = 64
_BATCH = 16384
_D = 2 * _EMBED

_info = plsc.get_sparse_core_info()
_NC, _NS = _info.num_cores, _info.num_subcores
_NW = _NC * _NS
_BPW = _BATCH // _NW
_CHUNK = 128
_NCHUNK = _BPW // _CHUNK


@functools.partial(
    pl.kernel,
    mesh=plsc.VectorSubcoreMesh(core_axis_name="c", subcore_axis_name="s"),
    out_type=jax.ShapeDtypeStruct((_BATCH, _D), jnp.float32),
    scratch_types=[
        pltpu.VMEM((_NCHUNK, _CHUNK), jnp.int32),
        pltpu.VMEM((_BPW, _D), jnp.float32),
        pltpu.SemaphoreType.DMA,
    ],
)
def _sc_gather(table_hbm, idx_hbm, out_hbm, idx_v, rows_v, sem):
    wid = lax.axis_index("s") * _NC + lax.axis_index("c")
    pltpu.sync_copy(idx_hbm.at[wid], idx_v)
    copies = [
        pltpu.async_copy(
            table_hbm.at[idx_v.at[j]],
            rows_v.at[pl.ds(j * _CHUNK, _CHUNK)],
            sem,
        )
        for j in range(_NCHUNK)
    ]
    for c in copies:
        c.wait()
    pltpu.sync_copy(rows_v, out_hbm.at[pl.ds(wid * _BPW, _BPW)])


def _tc_split_exp(g):
    def body(g_ref, mean_ref, var_ref):
        mean_ref[...] = g_ref[:, :_EMBED]
        var_ref[...] = jnp.exp(g_ref[:, _EMBED:] * 2.0)

    rows = 1024
    return pl.pallas_call(
        body,
        out_shape=(
            jax.ShapeDtypeStruct((_BATCH, _EMBED), jnp.float32),
            jax.ShapeDtypeStruct((_BATCH, _EMBED), jnp.float32),
        ),
        grid=(_BATCH // rows,),
        in_specs=[pl.BlockSpec((rows, _D), lambda i: (i, 0))],
        out_specs=(
            pl.BlockSpec((rows, _EMBED), lambda i: (i, 0)),
            pl.BlockSpec((rows, _EMBED), lambda i: (i, 0)),
        ),
    )(g)


def kernel(indices, W_mean, W_log_var):
    table = jnp.concatenate([W_mean, W_log_var], axis=0).T
    idx3 = indices.astype(jnp.int32).reshape(_NW, _NCHUNK, _CHUNK)
    g = _sc_gather(table, idx3)
    mean, var = _tc_split_exp(g)
    return mean, var


# speedup vs baseline: 1.1175x; 1.1175x over previous
import functools

import jax
import jax.numpy as jnp
from jax import lax
from jax.experimental import pallas as pl
from jax.experimental.pallas import tpu as pltpu
from jax.experimental.pallas import tpu_sc as plsc

_VOCAB = 100000
_EMBED = 64
_BATCH = 16384

_info = plsc.get_sparse_core_info()
_NC, _NS = _info.num_cores, _info.num_subcores
_NW = _NC * _NS
_RPW = _EMBED // _NW
_OCHUNK = 2048
_UNROLL = 8


@functools.partial(
    pl.kernel,
    mesh=plsc.VectorSubcoreMesh(core_axis_name="c", subcore_axis_name="s"),
    compiler_params=pltpu.CompilerParams(needs_layout_passes=False),
    out_type=jax.ShapeDtypeStruct((_EMBED, _BATCH), jnp.float32),
    scratch_types=[
        pltpu.VMEM((_VOCAB,), jnp.float32),
        pltpu.VMEM((_BATCH,), jnp.int32),
        pltpu.VMEM((2, _OCHUNK), jnp.float32),
        pltpu.SemaphoreType.DMA,
        pltpu.SemaphoreType.DMA,
    ],
)
def _sc_rowgather(table_hbm, idx_hbm, out_hbm, row_v, idx_v, ob_v, isem, osem):
    wid = lax.axis_index("s") * _NC + lax.axis_index("c")
    icopy = pltpu.async_copy(idx_hbm, idx_v, isem)
    icopy.wait()
    for r in range(_RPW):
        row = wid * _RPW + r
        pltpu.sync_copy(table_hbm.at[row], row_v)
        for c in range(_BATCH // _OCHUNK):
            buf = (c + r * (_BATCH // _OCHUNK)) % 2
            if c + r > 0:
                pass

            def body(i, _):
                iv = idx_v[pl.ds(c * _OCHUNK + i * 16, 16)]
                g = plsc.load_gather(row_v, [iv])
                ob_v[buf, pl.ds(i * 16, 16)] = g
                return _

            lax.fori_loop(0, _OCHUNK // 16, body, 0, unroll=_UNROLL)
            pltpu.async_copy(
                ob_v.at[buf], out_hbm.at[row, pl.ds(c * _OCHUNK, _OCHUNK)], osem
            ).wait()


def _tc_transpose_exp(gm, gv):
    def body(m_ref, v_ref, mean_ref, var_ref):
        mean_ref[...] = m_ref[...].T
        var_ref[...] = jnp.exp(v_ref[...].T * 2.0)

    cols = 512
    return pl.pallas_call(
        body,
        out_shape=(
            jax.ShapeDtypeStruct((_BATCH, _EMBED), jnp.float32),
            jax.ShapeDtypeStruct((_BATCH, _EMBED), jnp.float32),
        ),
        grid=(_BATCH // cols,),
        in_specs=[
            pl.BlockSpec((_EMBED, cols), lambda i: (0, i)),
            pl.BlockSpec((_EMBED, cols), lambda i: (0, i)),
        ],
        out_specs=(
            pl.BlockSpec((cols, _EMBED), lambda i: (i, 0)),
            pl.BlockSpec((cols, _EMBED), lambda i: (i, 0)),
        ),
    )(gm, gv)


def kernel(indices, W_mean, W_log_var):
    idx = indices.astype(jnp.int32)
    gm = _sc_rowgather(W_mean, idx)
    gv = _sc_rowgather(W_log_var, idx)
    mean, var = _tc_transpose_exp(gm, gv)
    return mean, var


# speedup vs baseline: 1.8893x; 1.6907x over previous
import functools

import jax
import jax.numpy as jnp
from jax import lax
from jax.experimental import pallas as pl
from jax.experimental.pallas import tpu as pltpu
from jax.experimental.pallas import tpu_sc as plsc

_VOCAB = 100000
_EMBED = 64
_BATCH = 16384

_info = plsc.get_sparse_core_info()
_NC, _NS = _info.num_cores, _info.num_subcores
_NW = _NC * _NS
_RPT = _EMBED // _NW
_OCHUNK = 2048
_UNROLL = 8


@functools.partial(
    pl.kernel,
    mesh=plsc.VectorSubcoreMesh(core_axis_name="c", subcore_axis_name="s"),
    compiler_params=pltpu.CompilerParams(needs_layout_passes=False),
    out_type=(
        jax.ShapeDtypeStruct((_EMBED, _BATCH), jnp.float32),
        jax.ShapeDtypeStruct((_EMBED, _BATCH), jnp.float32),
    ),
    scratch_types=[
        pltpu.VMEM((_VOCAB,), jnp.float32),
        pltpu.VMEM((_BATCH,), jnp.int32),
        pltpu.VMEM((2 * _OCHUNK,), jnp.float32),
        pltpu.SemaphoreType.DMA,
        pltpu.SemaphoreType.DMA,
    ],
)
def _sc_rowgather(wm_hbm, wlv_hbm, idx_hbm, om_hbm, olv_hbm, row_v, idx_v, ob_v, isem, osem):
    wid = lax.axis_index("s") * _NC + lax.axis_index("c")
    pltpu.async_copy(idx_hbm, idx_v, isem).wait()
    pending = []
    for tbl, out in ((wm_hbm, om_hbm), (wlv_hbm, olv_hbm)):
        for r in range(_RPT):
            row = wid * _RPT + r
            pltpu.sync_copy(tbl.at[row], row_v)
            for c in range(_BATCH // _OCHUNK):
                buf = len(pending) % 2
                if len(pending) >= 2:
                    pending[-2].wait()

                @plsc.parallel_loop(0, _OCHUNK, 16, unroll=_UNROLL)
                def body(i):
                    iv = idx_v[pl.ds(c * _OCHUNK + i, 16)]
                    g = plsc.load_gather(row_v, [iv])
                    ob_v[pl.ds(buf * _OCHUNK + i, 16)] = g
                pending.append(
                    pltpu.async_copy(
                        ob_v.at[pl.ds(buf * _OCHUNK, _OCHUNK)],
                        out.at[row, pl.ds(c * _OCHUNK, _OCHUNK)],
                        osem,
                    )
                )
    pending[-2].wait()
    pending[-1].wait()


def _tc_transpose_exp(gm, gv):
    def body(m_ref, v_ref, mean_ref, var_ref):
        ii = lax.broadcasted_iota(jnp.int32, (_EMBED, _EMBED), 0)
        jj = lax.broadcasted_iota(jnp.int32, (_EMBED, _EMBED), 1)
        eye = (ii == jj).astype(jnp.float32)
        dn = (((0,), (0,)), ((), ()))
        mean_ref[...] = lax.dot_general(
            m_ref[...], eye, dn, preferred_element_type=jnp.float32
        )
        vt = lax.dot_general(v_ref[...], eye, dn, preferred_element_type=jnp.float32)
        var_ref[...] = jnp.exp(vt * 2.0)

    cols = 2048
    return pl.pallas_call(
        body,
        out_shape=(
            jax.ShapeDtypeStruct((_BATCH, _EMBED), jnp.float32),
            jax.ShapeDtypeStruct((_BATCH, _EMBED), jnp.float32),
        ),
        grid=(_BATCH // cols,),
        in_specs=[
            pl.BlockSpec((_EMBED, cols), lambda i: (0, i)),
            pl.BlockSpec((_EMBED, cols), lambda i: (0, i)),
        ],
        out_specs=(
            pl.BlockSpec((cols, _EMBED), lambda i: (i, 0)),
            pl.BlockSpec((cols, _EMBED), lambda i: (i, 0)),
        ),
    )(gm, gv)


def kernel(indices, W_mean, W_log_var):
    idx = indices.astype(jnp.int32)
    gm, gv = _sc_rowgather(W_mean, W_log_var, idx)
    mean, var = _tc_transpose_exp(gm, gv)
    return mean, var


# speedup vs baseline: 2.8260x; 1.4958x over previous
import functools

import jax
import jax.numpy as jnp
from jax import lax
from jax.experimental import pallas as pl
from jax.experimental.pallas import tpu as pltpu
from jax.experimental.pallas import tpu_sc as plsc

_VOCAB = 100000
_EMBED = 64
_BATCH = 16384

_info = plsc.get_sparse_core_info()
_NC, _NS = _info.num_cores, _info.num_subcores
_NW = _NC * _NS
_RPT = _EMBED // _NW
_OCHUNK = 2048
_UNROLL = 8


@functools.partial(
    pl.kernel,
    mesh=plsc.VectorSubcoreMesh(core_axis_name="c", subcore_axis_name="s"),
    compiler_params=pltpu.CompilerParams(needs_layout_passes=False),
    out_type=(
        jax.ShapeDtypeStruct((_EMBED, _BATCH), jnp.float32),
        jax.ShapeDtypeStruct((_EMBED, _BATCH), jnp.float32),
    ),
    scratch_types=[
        pltpu.VMEM((_VOCAB,), jnp.float32),
        pltpu.VMEM((_BATCH,), jnp.int32),
        pltpu.VMEM((2 * _OCHUNK,), jnp.float32),
        pltpu.SemaphoreType.DMA,
        pltpu.SemaphoreType.DMA,
    ],
)
def _sc_rowgather(wm_hbm, wlv_hbm, idx_hbm, om_hbm, olv_hbm, row_v, idx_v, ob_v, isem, osem):
    wid = lax.axis_index("s") * _NC + lax.axis_index("c")
    icopy = pltpu.async_copy(idx_hbm, idx_v, isem)
    pending = []
    first = True
    for tbl, out, is_var in ((wm_hbm, om_hbm, False), (wlv_hbm, olv_hbm, True)):
        for r in range(_RPT):
            row = wid * _RPT + r
            pltpu.sync_copy(tbl.at[row], row_v)
            if first:
                icopy.wait()
                first = False
            for c in range(_BATCH // _OCHUNK):
                buf = len(pending) % 2
                if len(pending) >= 2:
                    pending[-2].wait()

                @plsc.parallel_loop(0, _OCHUNK, 16, unroll=_UNROLL)
                def body(i):
                    iv = idx_v[pl.ds(c * _OCHUNK + i, 16)]
                    g = plsc.load_gather(row_v, [iv])
                    if is_var:
                        g = jnp.exp(g * 2.0)
                    ob_v[pl.ds(buf * _OCHUNK + i, 16)] = g

                pending.append(
                    pltpu.async_copy(
                        ob_v.at[pl.ds(buf * _OCHUNK, _OCHUNK)],
                        out.at[row, pl.ds(c * _OCHUNK, _OCHUNK)],
                        osem,
                    )
                )
    pending[-2].wait()
    pending[-1].wait()


def kernel(indices, W_mean, W_log_var):
    idx = indices.astype(jnp.int32)
    gm, gv = _sc_rowgather(W_mean, W_log_var, idx)
    return gm.T, gv.T


# speedup vs baseline: 6.6029x; 2.3365x over previous
import functools

import jax
import jax.numpy as jnp
from jax import lax
from jax.experimental import pallas as pl
from jax.experimental.pallas import tpu as pltpu
from jax.experimental.pallas import tpu_sc as plsc

_VOCAB = 100000
_EMBED = 64
_BATCH = 16384

_info = plsc.get_sparse_core_info()
_NC, _NS = _info.num_cores, _info.num_subcores
_NW = _NC * _NS
_RPT = _EMBED // _NW
_OCHUNK = 2048
_UNROLL = 16


@functools.partial(
    pl.kernel,
    mesh=plsc.VectorSubcoreMesh(core_axis_name="c", subcore_axis_name="s"),
    compiler_params=pltpu.CompilerParams(needs_layout_passes=False),
    out_type=(
        jax.ShapeDtypeStruct((_EMBED, _BATCH), jnp.float32),
        jax.ShapeDtypeStruct((_EMBED, _BATCH), jnp.float32),
    ),
    scratch_types=[
        pltpu.VMEM((_VOCAB,), jnp.float32),
        pltpu.VMEM((_BATCH,), jnp.int32),
        pltpu.VMEM((4 * _OCHUNK,), jnp.float32),
        pltpu.SemaphoreType.DMA,
        pltpu.SemaphoreType.DMA,
    ],
)
def _sc_rowgather(wm_hbm, wlv_hbm, idx_hbm, om_hbm, olv_hbm, row_v, idx_v, ob_v, isem, osem):
    wid = lax.axis_index("s") * _NC + lax.axis_index("c")
    icopy = pltpu.async_copy(idx_hbm, idx_v, isem)
    pending = []
    first = True
    for tbl, out, is_var in ((wm_hbm, om_hbm, False), (wlv_hbm, olv_hbm, True)):
        for r in range(_RPT):
            row = wid * _RPT + r
            pltpu.sync_copy(tbl.at[row], row_v)
            if first:
                icopy.wait()
                first = False
            for c in range(_BATCH // _OCHUNK):
                buf = len(pending) % 4
                if len(pending) >= 4:
                    pending[-4].wait()

                @plsc.parallel_loop(0, _OCHUNK, 16, unroll=_UNROLL)
                def body(i):
                    iv = idx_v[pl.ds(c * _OCHUNK + i, 16)]
                    g = plsc.load_gather(row_v, [iv])
                    if is_var:
                        g = jnp.exp(g * 2.0)
                    ob_v[pl.ds(buf * _OCHUNK + i, 16)] = g

                pending.append(
                    pltpu.async_copy(
                        ob_v.at[pl.ds(buf * _OCHUNK, _OCHUNK)],
                        out.at[row, pl.ds(c * _OCHUNK, _OCHUNK)],
                        osem,
                    )
                )
    pending[-2].wait()
    pending[-1].wait()


def kernel(indices, W_mean, W_log_var):
    idx = indices.astype(jnp.int32)
    gm, gv = _sc_rowgather(W_mean, W_log_var, idx)
    return gm.T, gv.T
